# baseline (speedup 1.0000x reference)
import functools

import jax
import jax.numpy as jnp
from jax.experimental import pallas as pl
from jax.experimental.pallas import tpu as pltpu

N, D, H, K = 10000, 128, 64, 16
BLOCK_N = 400


def _pre_kernel(x_ref, w1_ref, bn_ref, h_ref, whh_ref, bias_ref,
                xw_ref, hgh_ref):
    gamma, rvar = bn_ref[0, :], bn_ref[3, :]
    scale = gamma * jax.lax.rsqrt(rvar + 1e-5)
    xw_ref[...] = jnp.dot(x_ref[...], w1_ref[...],
                          preferred_element_type=jnp.float32) * scale
    h = h_ref[...]
    gh = jnp.dot(h, whh_ref[...], preferred_element_type=jnp.float32) + bias_ref[1, :]
    hgh_ref[...] = jnp.concatenate([h, gh], axis=1)


def _main_kernel(xw_ref, a_ref, hgh_ref, bn_ref, wih_ref, bias_ref,
                 out_y_ref, out_h_ref, acc_ref, hs_ref, sem_y, sem_h):
    i = pl.program_id(0)
    nsteps = pl.num_programs(0)

    gamma, beta, rmean, rvar, b1 = (bn_ref[0, :], bn_ref[1, :],
                                    bn_ref[2, :], bn_ref[3, :], bn_ref[4, :])
    scale = gamma * jax.lax.rsqrt(rvar + 1e-5)
    shift = (b1 - rmean) * scale + beta
    alpha = 1.6732632423543772
    lam = 1.0507009873554805

    rows = pl.ds(i * BLOCK_N, BLOCK_N)
    y = jnp.dot(a_ref[...], xw_ref[...],
                preferred_element_type=jnp.float32) + shift
    y = lam * jnp.where(y > 0, y, alpha * (jnp.exp(y) - 1.0))
    hgh = hgh_ref[rows, :]
    h = hgh[:, 0:K]
    gi = jnp.dot(y, wih_ref[...], preferred_element_type=jnp.float32) + bias_ref[0, :]
    r = jax.nn.sigmoid(gi[:, 0:K] + hgh[:, K:2 * K])
    z = jax.nn.sigmoid(gi[:, K:2 * K] + hgh[:, 2 * K:3 * K])
    n = jnp.tanh(gi[:, 2 * K:3 * K] + r * hgh[:, 3 * K:4 * K])
    hs_ref[rows, :] = n + z * (h - n)
    acc_ref[rows, :] = y

    @pl.when(i == nsteps - 1)
    def _epilogue():
        cp_y = pltpu.make_async_copy(acc_ref, out_y_ref, sem_y)
        cp_h = pltpu.make_async_copy(hs_ref, out_h_ref, sem_h)
        cp_y.start()
        cp_h.start()
        cp_y.wait()
        cp_h.wait()


@functools.partial(jax.jit, static_argnames=("interpret",))
def _run(x, a, h0, W1, b1, gamma, beta, rmean, rvar, Wih, Whh, bih, bhh,
         interpret=False):
    bn = jnp.stack([gamma, beta, rmean, rvar, b1], axis=0)
    bias = jnp.stack([bih, bhh], axis=0)

    xw, hgh = pl.pallas_call(
        _pre_kernel,
        out_shape=[
            jax.ShapeDtypeStruct((N, H), jnp.float32),
            jax.ShapeDtypeStruct((N, 4 * K), jnp.float32),
        ],
        interpret=interpret,
    )(x, W1, bn, h0, Whh.T, bias)

    grid = (N // BLOCK_N,)
    row = lambda i: (i, 0)
    rep = lambda i: (0, 0)
    out_y, out_h = pl.pallas_call(
        _main_kernel,
        grid=grid,
        in_specs=[
            pl.BlockSpec((N, H), rep),
            pl.BlockSpec((BLOCK_N, N), row),
            pl.BlockSpec((N, 4 * K), rep),
            pl.BlockSpec((5, H), rep),
            pl.BlockSpec((H, 3 * K), rep),
            pl.BlockSpec((2, 3 * K), rep),
        ],
        out_specs=[
            pl.BlockSpec(memory_space=pltpu.MemorySpace.HBM),
            pl.BlockSpec(memory_space=pltpu.MemorySpace.HBM),
        ],
        out_shape=[
            jax.ShapeDtypeStruct((N, H), jnp.float32),
            jax.ShapeDtypeStruct((N, K), jnp.float32),
        ],
        scratch_shapes=[
            pltpu.VMEM((N, H), jnp.float32),
            pltpu.VMEM((N, K), jnp.float32),
            pltpu.SemaphoreType.DMA,
            pltpu.SemaphoreType.DMA,
        ],
        compiler_params=pltpu.CompilerParams(
            dimension_semantics=("arbitrary",),
        ),
        interpret=interpret,
    )(xw, a, hgh, bn, Wih.T, bias)
    return out_y, out_h


def kernel(features_list, norm_adjacency_list, adjacency_list,
           init_assignments, W1, b1, gamma, beta, rmean, rvar,
           Wih, Whh, bih, bhh, interpret=False):
    x = features_list[0]
    a = norm_adjacency_list[0]
    out_y, out_h = _run(x, a, init_assignments, W1, b1, gamma, beta,
                        rmean, rvar, Wih, Whh, bih, bhh,
                        interpret=interpret)
    return (out_h[None], out_y[None])


# speedup vs baseline: 1.0340x; 1.0340x over previous
import functools

import jax
import jax.numpy as jnp
from jax.experimental import pallas as pl
from jax.experimental.pallas import tpu as pltpu

N, D, H, K = 10000, 128, 64, 16
BLOCK_N = 400


def _pre_kernel(x_ref, w1_ref, bn_ref, xw_ref):
    gamma, rvar = bn_ref[0, :], bn_ref[3, :]
    scale = gamma * jax.lax.rsqrt(rvar + 1e-5)
    xw_ref[...] = jnp.dot(x_ref[...], w1_ref[...],
                          preferred_element_type=jnp.float32) * scale


def _main_kernel(xw_ref, a_ref, h_ref, bn_ref, wih_ref, whh_ref, bias_ref,
                 out_y_ref, out_h_ref):
    i = pl.program_id(0)

    gamma, beta, rmean, rvar, b1 = (bn_ref[0, :], bn_ref[1, :],
                                    bn_ref[2, :], bn_ref[3, :], bn_ref[4, :])
    scale = gamma * jax.lax.rsqrt(rvar + 1e-5)
    shift = (b1 - rmean) * scale + beta
    alpha = 1.6732632423543772
    lam = 1.0507009873554805

    rows = pl.ds(i * BLOCK_N, BLOCK_N)
    y = jnp.dot(a_ref[...], xw_ref[...],
                preferred_element_type=jnp.float32) + shift
    y = lam * jnp.where(y > 0, y, alpha * (jnp.exp(y) - 1.0))
    h = h_ref[rows, :]
    gi = jnp.dot(y, wih_ref[...], preferred_element_type=jnp.float32) + bias_ref[0, :]
    gh = jnp.dot(h, whh_ref[...], preferred_element_type=jnp.float32) + bias_ref[1, :]
    r = jax.nn.sigmoid(gi[:, 0:K] + gh[:, 0:K])
    z = jax.nn.sigmoid(gi[:, K:2 * K] + gh[:, K:2 * K])
    n = jnp.tanh(gi[:, 2 * K:3 * K] + r * gh[:, 2 * K:3 * K])
    out_h_ref[rows, :] = n + z * (h - n)
    out_y_ref[rows, :] = y


@functools.partial(jax.jit, static_argnames=("interpret",))
def _run(x, a, h0, W1, b1, gamma, beta, rmean, rvar, Wih, Whh, bih, bhh,
         interpret=False):
    bn = jnp.stack([gamma, beta, rmean, rvar, b1], axis=0)
    bias = jnp.stack([bih, bhh], axis=0)

    xw = pl.pallas_call(
        _pre_kernel,
        out_shape=jax.ShapeDtypeStruct((N, H), jnp.float32),
        interpret=interpret,
    )(x, W1, bn)

    grid = (N // BLOCK_N,)
    row = lambda i: (i, 0)
    rep = lambda i: (0, 0)
    out_y, out_h = pl.pallas_call(
        _main_kernel,
        grid=grid,
        in_specs=[
            pl.BlockSpec((N, H), rep),
            pl.BlockSpec((BLOCK_N, N), row),
            pl.BlockSpec((N, K), rep),
            pl.BlockSpec((5, H), rep),
            pl.BlockSpec((H, 3 * K), rep),
            pl.BlockSpec((K, 3 * K), rep),
            pl.BlockSpec((2, 3 * K), rep),
        ],
        out_specs=[
            pl.BlockSpec((N, H), rep),
            pl.BlockSpec((N, K), rep),
        ],
        out_shape=[
            jax.ShapeDtypeStruct((N, H), jnp.float32),
            jax.ShapeDtypeStruct((N, K), jnp.float32),
        ],
        compiler_params=pltpu.CompilerParams(
            dimension_semantics=("arbitrary",),
        ),
        interpret=interpret,
    )(xw, a, h0, bn, Wih.T, Whh.T, bias)
    return out_y, out_h


def kernel(features_list, norm_adjacency_list, adjacency_list,
           init_assignments, W1, b1, gamma, beta, rmean, rvar,
           Wih, Whh, bih, bhh, interpret=False):
    x = features_list[0]
    a = norm_adjacency_list[0]
    out_y, out_h = _run(x, a, init_assignments, W1, b1, gamma, beta,
                        rmean, rvar, Wih, Whh, bih, bhh,
                        interpret=interpret)
    return (out_h[None], out_y[None])
